# Initial kernel scaffold; baseline (speedup 1.0000x reference)
import jax
import jax.numpy as jnp
from jax.experimental import pallas as pl


def kernel(x, edge_index, W1, b1, Wmu, bmu, Wls, bls):
    raise NotImplementedError("write your pallas kernel here")


# speedup vs baseline: 20.1167x; 20.1167x over previous
import functools

import jax
import jax.numpy as jnp
from jax import lax
from jax.experimental import pallas as pl
from jax.experimental.pallas import tpu as pltpu
from jax.experimental.pallas import tpu_sc as plsc

_N = 10000
_E = 320000
_D_IN = 128
_D_OUT = 128
_D_HID = 256

_NC = 2
_NS = 16
_NP = 10240
_RPT = _NP // _NS

_C = 125
_R = _E // _C
_ESC = _E // _NC
_CH1 = _ESC // _NS // _C
_CH2 = _E // _NS // _C

_mesh = plsc.VectorSubcoreMesh(core_axis_name="c", subcore_axis_name="s")


@functools.partial(
    pl.kernel,
    out_type=jax.ShapeDtypeStruct((_NC * _NP, 16), jnp.float32),
    mesh=_mesh,
    scratch_types=[
        pltpu.VMEM((_CH1, _C), jnp.int32),
        pltpu.VMEM((128, 16), jnp.float32),
        pltpu.VMEM_SHARED((_NP, 16), jnp.float32),
    ],
)
def _sc_degree(dst_hbm, ones_hbm, out_hbm, dst_v, ones_v, acc):
    c = lax.axis_index("c")
    s = lax.axis_index("s")
    row0 = c * (_ESC // _C) + s * _CH1
    base = s * _RPT
    pltpu.sync_copy(dst_hbm.at[pl.ds(row0, _CH1)], dst_v)
    pltpu.sync_copy(ones_hbm.at[pl.ds(0, 128)], ones_v)
    pltpu.sync_copy(ones_hbm.at[pl.ds(base, _RPT)], acc.at[pl.ds(base, _RPT)])
    plsc.subcore_barrier()

    def body(k, carry):
        pltpu.sync_copy(ones_v.at[pl.ds(0, _C)], acc.at[dst_v.at[k]], add=True)
        return carry

    lax.fori_loop(0, _CH1, body, 0)
    plsc.subcore_barrier()
    pltpu.sync_copy(acc.at[pl.ds(base, _RPT)],
                    out_hbm.at[pl.ds(c * _NP + base, _RPT)])


@functools.partial(
    pl.kernel,
    out_type=jax.ShapeDtypeStruct((_NC * _NP, 128), jnp.float32),
    mesh=_mesh,
    scratch_types=[
        pltpu.VMEM((_CH1, _C), jnp.int32),
        pltpu.VMEM((_CH1, _C), jnp.int32),
        pltpu.VMEM((_C, 128), jnp.float32),
        pltpu.VMEM_SHARED((_NP, 128), jnp.float32),
    ],
)
def _sc_pass1(xs_hbm, src_hbm, dst_hbm, out_hbm, src_v, dst_v, rows_v, acc):
    c = lax.axis_index("c")
    s = lax.axis_index("s")
    row0 = c * (_ESC // _C) + s * _CH1
    base = s * _RPT
    pltpu.sync_copy(src_hbm.at[pl.ds(row0, _CH1)], src_v)
    pltpu.sync_copy(dst_hbm.at[pl.ds(row0, _CH1)], dst_v)
    pltpu.sync_copy(xs_hbm.at[pl.ds(base, _RPT)], acc.at[pl.ds(base, _RPT)])
    plsc.subcore_barrier()

    def body(k, carry):
        pltpu.sync_copy(xs_hbm.at[src_v.at[k]], rows_v)
        pltpu.sync_copy(rows_v, acc.at[dst_v.at[k]], add=True)
        return carry

    lax.fori_loop(0, _CH1, body, 0)
    plsc.subcore_barrier()
    pltpu.sync_copy(acc.at[pl.ds(base, _RPT)],
                    out_hbm.at[pl.ds(c * _NP + base, _RPT)])


@functools.partial(
    pl.kernel,
    out_type=jax.ShapeDtypeStruct((_NC * _NP, 128), jnp.float32),
    mesh=_mesh,
    scratch_types=[
        pltpu.VMEM((_CH1, _C), jnp.int32),
        pltpu.VMEM((_CH1, _C), jnp.int32),
        pltpu.VMEM((_C, 128), jnp.float32),
        pltpu.VMEM_SHARED((_NP, 128), jnp.float32),
    ],
)
def _sc_pass2(h0_hbm, h1_hbm, src_hbm, dst_hbm, out_hbm, src_v, dst_v, rows_v, acc):
    c = lax.axis_index("c")
    s = lax.axis_index("s")
    row0 = s * _CH2
    base = s * _RPT

    @pl.when(c == 0)
    def _():
        pltpu.sync_copy(h0_hbm.at[pl.ds(base, _RPT)], acc.at[pl.ds(base, _RPT)])

    @pl.when(c == 1)
    def _():
        pltpu.sync_copy(h1_hbm.at[pl.ds(base, _RPT)], acc.at[pl.ds(base, _RPT)])

    plsc.subcore_barrier()

    def body(k, carry):
        @pl.when(c == 0)
        def _():
            pltpu.sync_copy(h0_hbm.at[src_v.at[k]], rows_v)

        @pl.when(c == 1)
        def _():
            pltpu.sync_copy(h1_hbm.at[src_v.at[k]], rows_v)

        pltpu.sync_copy(rows_v, acc.at[dst_v.at[k]], add=True)
        return carry

    for g in range(_CH2 // _CH1):
        pltpu.sync_copy(src_hbm.at[pl.ds(row0 + g * _CH1, _CH1)], src_v)
        pltpu.sync_copy(dst_hbm.at[pl.ds(row0 + g * _CH1, _CH1)], dst_v)
        lax.fori_loop(0, _CH1, body, 0)
    plsc.subcore_barrier()
    pltpu.sync_copy(acc.at[pl.ds(base, _RPT)],
                    out_hbm.at[pl.ds(c * _NP + base, _RPT)])


_BN = 1024


def _tc_a(x, d0, d1):
    def body(x_ref, d0_ref, d1_ref, xs_ref, dis_ref):
        deg = d0_ref[:, 0:1] + d1_ref[:, 0:1] - 1.0
        dis = 1.0 / jnp.sqrt(deg)
        xs_ref[...] = x_ref[...] * dis
        dis_ref[...] = jnp.broadcast_to(dis, dis_ref.shape)

    return pl.pallas_call(
        body,
        grid=(_NP // _BN,),
        in_specs=[
            pl.BlockSpec((_BN, _D_IN), lambda i: (i, 0)),
            pl.BlockSpec((_BN, 16), lambda i: (i, 0)),
            pl.BlockSpec((_BN, 16), lambda i: (i, 0)),
        ],
        out_specs=[
            pl.BlockSpec((_BN, _D_IN), lambda i: (i, 0)),
            pl.BlockSpec((_BN, _D_IN), lambda i: (i, 0)),
        ],
        out_shape=[
            jax.ShapeDtypeStruct((_NP, _D_IN), jnp.float32),
            jax.ShapeDtypeStruct((_NP, _D_IN), jnp.float32),
        ],
    )(x, d0, d1)


def _tc_b(p0, p1, xs, dis, W1, b1):
    def body(p0_ref, p1_ref, xs_ref, dis_ref, w_ref, b_ref, h0_ref, h1_ref):
        agg = dis_ref[...] * (p0_ref[...] + p1_ref[...] - xs_ref[...])
        h = jnp.dot(agg, w_ref[...], preferred_element_type=jnp.float32,
                    precision="highest") + b_ref[...]
        h = jnp.where(h >= 0.0, h, 0.01 * h)
        h0_ref[...] = h[:, :128] * dis_ref[...]
        h1_ref[...] = h[:, 128:] * dis_ref[...]

    return pl.pallas_call(
        body,
        grid=(_NP // _BN,),
        in_specs=[
            pl.BlockSpec((_BN, 128), lambda i: (i, 0)),
            pl.BlockSpec((_BN, 128), lambda i: (i, 0)),
            pl.BlockSpec((_BN, 128), lambda i: (i, 0)),
            pl.BlockSpec((_BN, 128), lambda i: (i, 0)),
            pl.BlockSpec((_D_IN, _D_HID), lambda i: (0, 0)),
            pl.BlockSpec((1, _D_HID), lambda i: (0, 0)),
        ],
        out_specs=[
            pl.BlockSpec((_BN, 128), lambda i: (i, 0)),
            pl.BlockSpec((_BN, 128), lambda i: (i, 0)),
        ],
        out_shape=[
            jax.ShapeDtypeStruct((_NP, 128), jnp.float32),
            jax.ShapeDtypeStruct((_NP, 128), jnp.float32),
        ],
    )(p0, p1, xs, dis, W1, b1)


def _tc_c(q0, q1, dis, Wmu, bmu, Wls, bls):
    def body(q0_ref, q1_ref, dis_ref, wmu_ref, bmu_ref, wls_ref, bls_ref,
             mu_ref, ls_ref):
        a0 = dis_ref[...] * q0_ref[...]
        a1 = dis_ref[...] * q1_ref[...]
        mu_ref[...] = (
            jnp.dot(a0, wmu_ref[:128, :], preferred_element_type=jnp.float32,
                    precision="highest")
            + jnp.dot(a1, wmu_ref[128:, :], preferred_element_type=jnp.float32,
                      precision="highest")
            + bmu_ref[...])
        ls_ref[...] = (
            jnp.dot(a0, wls_ref[:128, :], preferred_element_type=jnp.float32,
                    precision="highest")
            + jnp.dot(a1, wls_ref[128:, :], preferred_element_type=jnp.float32,
                      precision="highest")
            + bls_ref[...])

    return pl.pallas_call(
        body,
        grid=(_NP // _BN,),
        in_specs=[
            pl.BlockSpec((_BN, 128), lambda i: (i, 0)),
            pl.BlockSpec((_BN, 128), lambda i: (i, 0)),
            pl.BlockSpec((_BN, 128), lambda i: (i, 0)),
            pl.BlockSpec((_D_HID, _D_OUT), lambda i: (0, 0)),
            pl.BlockSpec((1, _D_OUT), lambda i: (0, 0)),
            pl.BlockSpec((_D_HID, _D_OUT), lambda i: (0, 0)),
            pl.BlockSpec((1, _D_OUT), lambda i: (0, 0)),
        ],
        out_specs=[
            pl.BlockSpec((_BN, _D_OUT), lambda i: (i, 0)),
            pl.BlockSpec((_BN, _D_OUT), lambda i: (i, 0)),
        ],
        out_shape=[
            jax.ShapeDtypeStruct((_NP, _D_OUT), jnp.float32),
            jax.ShapeDtypeStruct((_NP, _D_OUT), jnp.float32),
        ],
    )(q0, q1, dis, Wmu, bmu, Wls, bls)


def kernel(x, edge_index, W1, b1, Wmu, bmu, Wls, bls):
    src = edge_index[0].reshape(_R, _C)
    dst = edge_index[1].reshape(_R, _C)
    ones = jnp.ones((_NP, 16), jnp.float32)
    x_pad = jnp.pad(x, ((0, _NP - _N), (0, 0)))

    d = _sc_degree(dst, ones)
    xs, dis = _tc_a(x_pad, d[:_NP], d[_NP:])
    p = _sc_pass1(xs, src, dst)
    h0, h1 = _tc_b(p[:_NP], p[_NP:], xs, dis, W1, b1.reshape(1, _D_HID))
    q = _sc_pass2(h0, h1, src, dst)
    mu, ls = _tc_c(q[:_NP], q[_NP:], dis, Wmu, bmu.reshape(1, _D_OUT),
                   Wls, bls.reshape(1, _D_OUT))
    return (mu[:_N], ls[:_N])


# speedup vs baseline: 22.9460x; 1.1406x over previous
import functools

import jax
import jax.numpy as jnp
from jax import lax
from jax.experimental import pallas as pl
from jax.experimental.pallas import tpu as pltpu
from jax.experimental.pallas import tpu_sc as plsc

_N = 10000
_E = 320000
_D_IN = 128
_D_OUT = 128
_D_HID = 256

_NC = 2
_NS = 16
_NP = 10240
_RPT = _NP // _NS

_C = 125
_R = _E // _C
_ESC = _E // _NC
_CH1 = _ESC // _NS // _C
_CH2 = _E // _NS // _C

_mesh = plsc.VectorSubcoreMesh(core_axis_name="c", subcore_axis_name="s")


@functools.partial(
    pl.kernel,
    out_type=jax.ShapeDtypeStruct((_NC * _NP, 16), jnp.float32),
    mesh=_mesh,
    scratch_types=[
        pltpu.VMEM((_CH1, _C), jnp.int32),
        pltpu.VMEM((128, 16), jnp.float32),
        pltpu.VMEM_SHARED((_NP, 16), jnp.float32),
    ],
)
def _sc_degree(dst_hbm, ones_hbm, out_hbm, dst_v, ones_v, acc):
    c = lax.axis_index("c")
    s = lax.axis_index("s")
    row0 = c * (_ESC // _C) + s * _CH1
    base = s * _RPT
    pltpu.sync_copy(dst_hbm.at[pl.ds(row0, _CH1)], dst_v)
    pltpu.sync_copy(ones_hbm.at[pl.ds(0, 128)], ones_v)
    pltpu.sync_copy(ones_hbm.at[pl.ds(base, _RPT)], acc.at[pl.ds(base, _RPT)])
    plsc.subcore_barrier()

    def body(k, carry):
        pltpu.sync_copy(ones_v.at[pl.ds(0, _C)], acc.at[dst_v.at[k]], add=True)
        return carry

    lax.fori_loop(0, _CH1, body, 0)
    plsc.subcore_barrier()
    pltpu.sync_copy(acc.at[pl.ds(base, _RPT)],
                    out_hbm.at[pl.ds(c * _NP + base, _RPT)])


_CR = 40
_PAIRS = _CR // 2


def _run_rounds(table, srcs, c, dst_hbm, src_v, dst_v, rowsA, rowsB,
                sga, sgb, ssa, ssb, acc, row0, nrounds):
    for r in range(nrounds):
        base_r = row0 + r * _CR
        if len(srcs) == 1:
            pltpu.sync_copy(srcs[0].at[pl.ds(base_r, _CR)], src_v)
        else:
            @pl.when(c == 0)
            def _():
                pltpu.sync_copy(srcs[0].at[pl.ds(base_r, _CR)], src_v)

            @pl.when(c == 1)
            def _():
                pltpu.sync_copy(srcs[1].at[pl.ds(base_r, _CR)], src_v)
        pltpu.sync_copy(dst_hbm.at[pl.ds(base_r, _CR)], dst_v)
        pltpu.async_copy(table.at[src_v.at[0]], rowsA, sga)
        pltpu.async_copy(table.at[src_v.at[1]], rowsB, sgb)

        def pair(j, carry):
            k0 = 2 * j
            pltpu.make_async_copy(table.at[src_v.at[0]], rowsA, sga).wait()
            pltpu.async_copy(rowsA, acc.at[dst_v.at[k0]], ssa, add=True)
            pltpu.make_async_copy(table.at[src_v.at[0]], rowsB, sgb).wait()
            pltpu.async_copy(rowsB, acc.at[dst_v.at[k0 + 1]], ssb, add=True)

            @pl.when(j < _PAIRS - 1)
            def _():
                pltpu.make_async_copy(rowsA, acc.at[dst_v.at[0]], ssa).wait()
                pltpu.async_copy(table.at[src_v.at[k0 + 2]], rowsA, sga)
                pltpu.make_async_copy(rowsB, acc.at[dst_v.at[0]], ssb).wait()
                pltpu.async_copy(table.at[src_v.at[k0 + 3]], rowsB, sgb)
            return carry

        lax.fori_loop(0, _PAIRS, pair, 0)
        pltpu.make_async_copy(rowsA, acc.at[dst_v.at[0]], ssa).wait()
        pltpu.make_async_copy(rowsB, acc.at[dst_v.at[0]], ssb).wait()


_PASS_SCRATCH = [
    pltpu.VMEM((_CR, _C), jnp.int32),
    pltpu.VMEM((_CR, _C), jnp.int32),
    pltpu.VMEM((_C, 128), jnp.float32),
    pltpu.VMEM((_C, 128), jnp.float32),
    pltpu.SemaphoreType.DMA,
    pltpu.SemaphoreType.DMA,
    pltpu.SemaphoreType.DMA,
    pltpu.SemaphoreType.DMA,
    pltpu.VMEM_SHARED((_NP, 128), jnp.float32),
]


@functools.partial(
    pl.kernel,
    out_type=jax.ShapeDtypeStruct((_NC * _NP, 128), jnp.float32),
    mesh=_mesh,
    scratch_types=_PASS_SCRATCH,
)
def _sc_pass1(xs_hbm, src_hbm, dst_hbm, out_hbm, src_v, dst_v, rowsA, rowsB,
              sga, sgb, ssa, ssb, acc):
    c = lax.axis_index("c")
    s = lax.axis_index("s")
    row0 = c * (_ESC // _C) + s * _CH1
    base = s * _RPT
    pltpu.sync_copy(xs_hbm.at[pl.ds(base, _RPT)], acc.at[pl.ds(base, _RPT)])
    plsc.subcore_barrier()
    _run_rounds(xs_hbm, (src_hbm,), c, dst_hbm, src_v, dst_v, rowsA, rowsB,
                sga, sgb, ssa, ssb, acc, row0, _CH1 // _CR)
    plsc.subcore_barrier()
    pltpu.sync_copy(acc.at[pl.ds(base, _RPT)],
                    out_hbm.at[pl.ds(c * _NP + base, _RPT)])


@functools.partial(
    pl.kernel,
    out_type=jax.ShapeDtypeStruct((_NC * _NP, 128), jnp.float32),
    mesh=_mesh,
    scratch_types=_PASS_SCRATCH,
)
def _sc_pass2(hcat_hbm, src_hbm, srcp_hbm, dst_hbm, out_hbm, src_v, dst_v,
              rowsA, rowsB, sga, sgb, ssa, ssb, acc):
    c = lax.axis_index("c")
    s = lax.axis_index("s")
    row0 = s * _CH2
    base = s * _RPT
    pltpu.sync_copy(hcat_hbm.at[pl.ds(c * _NP + base, _RPT)],
                    acc.at[pl.ds(base, _RPT)])
    plsc.subcore_barrier()
    _run_rounds(hcat_hbm, (src_hbm, srcp_hbm), c, dst_hbm, src_v, dst_v,
                rowsA, rowsB, sga, sgb, ssa, ssb, acc, row0, _CH2 // _CR)
    plsc.subcore_barrier()
    pltpu.sync_copy(acc.at[pl.ds(base, _RPT)],
                    out_hbm.at[pl.ds(c * _NP + base, _RPT)])


_BN = 1024


def _tc_a(x, d0, d1):
    def body(x_ref, d0_ref, d1_ref, xs_ref, dis_ref):
        deg = d0_ref[:, 0:1] + d1_ref[:, 0:1] - 1.0
        dis = 1.0 / jnp.sqrt(deg)
        xs_ref[...] = x_ref[...] * dis
        dis_ref[...] = jnp.broadcast_to(dis, dis_ref.shape)

    return pl.pallas_call(
        body,
        grid=(_NP // _BN,),
        in_specs=[
            pl.BlockSpec((_BN, _D_IN), lambda i: (i, 0)),
            pl.BlockSpec((_BN, 16), lambda i: (i, 0)),
            pl.BlockSpec((_BN, 16), lambda i: (i, 0)),
        ],
        out_specs=[
            pl.BlockSpec((_BN, _D_IN), lambda i: (i, 0)),
            pl.BlockSpec((_BN, _D_IN), lambda i: (i, 0)),
        ],
        out_shape=[
            jax.ShapeDtypeStruct((_NP, _D_IN), jnp.float32),
            jax.ShapeDtypeStruct((_NP, _D_IN), jnp.float32),
        ],
    )(x, d0, d1)


def _tc_b(p0, p1, xs, dis, W1, b1):
    def body(p0_ref, p1_ref, xs_ref, dis_ref, w_ref, b_ref, h0_ref, h1_ref):
        agg = dis_ref[...] * (p0_ref[...] + p1_ref[...] - xs_ref[...])
        h = jnp.dot(agg, w_ref[...], preferred_element_type=jnp.float32,
                    precision="highest") + b_ref[...]
        h = jnp.where(h >= 0.0, h, 0.01 * h)
        h0_ref[...] = h[:, :128] * dis_ref[...]
        h1_ref[...] = h[:, 128:] * dis_ref[...]

    return pl.pallas_call(
        body,
        grid=(_NP // _BN,),
        in_specs=[
            pl.BlockSpec((_BN, 128), lambda i: (i, 0)),
            pl.BlockSpec((_BN, 128), lambda i: (i, 0)),
            pl.BlockSpec((_BN, 128), lambda i: (i, 0)),
            pl.BlockSpec((_BN, 128), lambda i: (i, 0)),
            pl.BlockSpec((_D_IN, _D_HID), lambda i: (0, 0)),
            pl.BlockSpec((1, _D_HID), lambda i: (0, 0)),
        ],
        out_specs=[
            pl.BlockSpec((_BN, 128), lambda i: (i, 0)),
            pl.BlockSpec((_BN, 128), lambda i: (i, 0)),
        ],
        out_shape=[
            jax.ShapeDtypeStruct((_NP, 128), jnp.float32),
            jax.ShapeDtypeStruct((_NP, 128), jnp.float32),
        ],
    )(p0, p1, xs, dis, W1, b1)


def _tc_c(q0, q1, dis, Wmu, bmu, Wls, bls):
    def body(q0_ref, q1_ref, dis_ref, wmu_ref, bmu_ref, wls_ref, bls_ref,
             mu_ref, ls_ref):
        a0 = dis_ref[...] * q0_ref[...]
        a1 = dis_ref[...] * q1_ref[...]
        mu_ref[...] = (
            jnp.dot(a0, wmu_ref[:128, :], preferred_element_type=jnp.float32,
                    precision="highest")
            + jnp.dot(a1, wmu_ref[128:, :], preferred_element_type=jnp.float32,
                      precision="highest")
            + bmu_ref[...])
        ls_ref[...] = (
            jnp.dot(a0, wls_ref[:128, :], preferred_element_type=jnp.float32,
                    precision="highest")
            + jnp.dot(a1, wls_ref[128:, :], preferred_element_type=jnp.float32,
                      precision="highest")
            + bls_ref[...])

    return pl.pallas_call(
        body,
        grid=(_NP // _BN,),
        in_specs=[
            pl.BlockSpec((_BN, 128), lambda i: (i, 0)),
            pl.BlockSpec((_BN, 128), lambda i: (i, 0)),
            pl.BlockSpec((_BN, 128), lambda i: (i, 0)),
            pl.BlockSpec((_D_HID, _D_OUT), lambda i: (0, 0)),
            pl.BlockSpec((1, _D_OUT), lambda i: (0, 0)),
            pl.BlockSpec((_D_HID, _D_OUT), lambda i: (0, 0)),
            pl.BlockSpec((1, _D_OUT), lambda i: (0, 0)),
        ],
        out_specs=[
            pl.BlockSpec((_BN, _D_OUT), lambda i: (i, 0)),
            pl.BlockSpec((_BN, _D_OUT), lambda i: (i, 0)),
        ],
        out_shape=[
            jax.ShapeDtypeStruct((_NP, _D_OUT), jnp.float32),
            jax.ShapeDtypeStruct((_NP, _D_OUT), jnp.float32),
        ],
    )(q0, q1, dis, Wmu, bmu, Wls, bls)


def kernel(x, edge_index, W1, b1, Wmu, bmu, Wls, bls):
    src = edge_index[0].reshape(_R, _C)
    dst = edge_index[1].reshape(_R, _C)
    ones = jnp.ones((_NP, 16), jnp.float32)
    x_pad = jnp.pad(x, ((0, _NP - _N), (0, 0)))

    d = _sc_degree(dst, ones)
    xs, dis = _tc_a(x_pad, d[:_NP], d[_NP:])
    p = _sc_pass1(xs, src, dst)
    h0, h1 = _tc_b(p[:_NP], p[_NP:], xs, dis, W1, b1.reshape(1, _D_HID))
    hcat = jnp.concatenate([h0, h1], axis=0)
    srcp = (edge_index[0] + _NP).reshape(_R, _C)
    q = _sc_pass2(hcat, src, srcp, dst)
    mu, ls = _tc_c(q[:_NP], q[_NP:], dis, Wmu, bmu.reshape(1, _D_OUT),
                   Wls, bls.reshape(1, _D_OUT))
    return (mu[:_N], ls[:_N])


# speedup vs baseline: 24.9058x; 1.0854x over previous
import functools

import jax
import jax.numpy as jnp
from jax import lax
from jax.experimental import pallas as pl
from jax.experimental.pallas import tpu as pltpu
from jax.experimental.pallas import tpu_sc as plsc

_N = 10000
_E = 320000
_D_IN = 128
_D_OUT = 128
_D_HID = 256

_NC = 2
_NS = 16
_NP = 10240
_RPT = _NP // _NS

_C = 125
_R = _E // _C
_ESC = _E // _NC
_CH1 = _ESC // _NS // _C
_CH2 = _E // _NS // _C

_mesh = plsc.VectorSubcoreMesh(core_axis_name="c", subcore_axis_name="s")


@functools.partial(
    pl.kernel,
    out_type=jax.ShapeDtypeStruct((_NC * _NP, 16), jnp.float32),
    mesh=_mesh,
    scratch_types=[
        pltpu.VMEM((_CH1, _C), jnp.int32),
        pltpu.VMEM((128, 16), jnp.float32),
        pltpu.SemaphoreType.DMA,
        pltpu.VMEM_SHARED((_NP, 16), jnp.float32),
    ],
)
def _sc_degree(dst_hbm, ones_hbm, out_hbm, dst_v, ones_v, sem, acc):
    c = lax.axis_index("c")
    s = lax.axis_index("s")
    row0 = c * (_ESC // _C) + s * _CH1
    base = s * _RPT
    pltpu.sync_copy(dst_hbm.at[pl.ds(row0, _CH1)], dst_v)
    pltpu.sync_copy(ones_hbm.at[pl.ds(0, 128)], ones_v)
    pltpu.sync_copy(ones_hbm.at[pl.ds(base, _RPT)], acc.at[pl.ds(base, _RPT)])
    plsc.subcore_barrier()

    def issue(k):
        pltpu.async_copy(ones_v.at[pl.ds(0, _C)], acc.at[dst_v.at[k]], sem,
                         add=True)

    def wait_one():
        pltpu.make_async_copy(ones_v.at[pl.ds(0, _C)], acc.at[dst_v.at[0]],
                              sem).wait()

    _OUT = 4
    for k in range(_OUT):
        issue(k)

    def body(k, carry):
        wait_one()
        issue(k + _OUT)
        return carry

    lax.fori_loop(0, _CH1 - _OUT, body, 0)
    for _ in range(_OUT):
        wait_one()
    plsc.subcore_barrier()
    pltpu.sync_copy(acc.at[pl.ds(base, _RPT)],
                    out_hbm.at[pl.ds(c * _NP + base, _RPT)])


_CR = 40
_PAIRS = _CR // 2


def _run_rounds(tables, c, src_hbm, dst_hbm, src_v, dst_v, rowsA, rowsB,
                sga, sgb, ssa, ssb, acc, row0, nrounds):
    def issue_gather(k, buf, sem):
        if len(tables) == 1:
            pltpu.async_copy(tables[0].at[src_v.at[k]], buf, sem)
        else:
            @pl.when(c == 0)
            def _():
                pltpu.async_copy(tables[0].at[src_v.at[k]], buf, sem)

            @pl.when(c == 1)
            def _():
                pltpu.async_copy(tables[1].at[src_v.at[k]], buf, sem)

    def wait_gather(buf, sem):
        pltpu.make_async_copy(tables[0].at[src_v.at[0]], buf, sem).wait()

    def wait_scatter(buf, sem):
        pltpu.make_async_copy(buf, acc.at[dst_v.at[0]], sem).wait()

    for r in range(nrounds):
        base_r = row0 + r * _CR
        pltpu.sync_copy(src_hbm.at[pl.ds(base_r, _CR)], src_v)
        pltpu.sync_copy(dst_hbm.at[pl.ds(base_r, _CR)], dst_v)
        issue_gather(0, rowsA, sga)
        issue_gather(1, rowsB, sgb)

        def pair(j, carry):
            k0 = 2 * j
            wait_gather(rowsA, sga)
            pltpu.async_copy(rowsA, acc.at[dst_v.at[k0]], ssa, add=True)
            wait_gather(rowsB, sgb)
            pltpu.async_copy(rowsB, acc.at[dst_v.at[k0 + 1]], ssb, add=True)

            @pl.when(j < _PAIRS - 1)
            def _():
                wait_scatter(rowsA, ssa)
                issue_gather(k0 + 2, rowsA, sga)
                wait_scatter(rowsB, ssb)
                issue_gather(k0 + 3, rowsB, sgb)
            return carry

        lax.fori_loop(0, _PAIRS, pair, 0)
        wait_scatter(rowsA, ssa)
        wait_scatter(rowsB, ssb)


_PASS_SCRATCH = [
    pltpu.VMEM((_CR, _C), jnp.int32),
    pltpu.VMEM((_CR, _C), jnp.int32),
    pltpu.VMEM((_C, 128), jnp.float32),
    pltpu.VMEM((_C, 128), jnp.float32),
    pltpu.SemaphoreType.DMA,
    pltpu.SemaphoreType.DMA,
    pltpu.SemaphoreType.DMA,
    pltpu.SemaphoreType.DMA,
    pltpu.VMEM_SHARED((_NP, 128), jnp.float32),
]


@functools.partial(
    pl.kernel,
    out_type=jax.ShapeDtypeStruct((_NC * _NP, 128), jnp.float32),
    mesh=_mesh,
    scratch_types=_PASS_SCRATCH,
)
def _sc_pass1(xs_hbm, src_hbm, dst_hbm, out_hbm, src_v, dst_v, rowsA, rowsB,
              sga, sgb, ssa, ssb, acc):
    c = lax.axis_index("c")
    s = lax.axis_index("s")
    row0 = c * (_ESC // _C) + s * _CH1
    base = s * _RPT
    pltpu.sync_copy(xs_hbm.at[pl.ds(base, _RPT)], acc.at[pl.ds(base, _RPT)])
    plsc.subcore_barrier()
    _run_rounds((xs_hbm,), c, src_hbm, dst_hbm, src_v, dst_v, rowsA, rowsB,
                sga, sgb, ssa, ssb, acc, row0, _CH1 // _CR)
    plsc.subcore_barrier()
    pltpu.sync_copy(acc.at[pl.ds(base, _RPT)],
                    out_hbm.at[pl.ds(c * _NP + base, _RPT)])


@functools.partial(
    pl.kernel,
    out_type=jax.ShapeDtypeStruct((_NC * _NP, 128), jnp.float32),
    mesh=_mesh,
    scratch_types=_PASS_SCRATCH,
)
def _sc_pass2(h0_hbm, h1_hbm, src_hbm, dst_hbm, out_hbm, src_v, dst_v,
              rowsA, rowsB, sga, sgb, ssa, ssb, acc):
    c = lax.axis_index("c")
    s = lax.axis_index("s")
    row0 = s * _CH2
    base = s * _RPT

    @pl.when(c == 0)
    def _():
        pltpu.sync_copy(h0_hbm.at[pl.ds(base, _RPT)], acc.at[pl.ds(base, _RPT)])

    @pl.when(c == 1)
    def _():
        pltpu.sync_copy(h1_hbm.at[pl.ds(base, _RPT)], acc.at[pl.ds(base, _RPT)])

    plsc.subcore_barrier()
    _run_rounds((h0_hbm, h1_hbm), c, src_hbm, dst_hbm, src_v, dst_v,
                rowsA, rowsB, sga, sgb, ssa, ssb, acc, row0, _CH2 // _CR)
    plsc.subcore_barrier()
    pltpu.sync_copy(acc.at[pl.ds(base, _RPT)],
                    out_hbm.at[pl.ds(c * _NP + base, _RPT)])


_BN = 1024


def _tc_a(x, d):
    def body(x_ref, d0_ref, d1_ref, xs_ref, dis_ref):
        deg = d0_ref[:, 0:1] + d1_ref[:, 0:1] - 1.0
        dis = 1.0 / jnp.sqrt(deg)
        xs_ref[...] = x_ref[...] * dis
        dis_ref[...] = jnp.broadcast_to(dis, dis_ref.shape)

    nb = _NP // _BN
    return pl.pallas_call(
        body,
        grid=(nb,),
        in_specs=[
            pl.BlockSpec((_BN, _D_IN), lambda i: (i, 0)),
            pl.BlockSpec((_BN, 16), lambda i: (i, 0)),
            pl.BlockSpec((_BN, 16), lambda i: (i + nb, 0)),
        ],
        out_specs=[
            pl.BlockSpec((_BN, _D_IN), lambda i: (i, 0)),
            pl.BlockSpec((_BN, _D_IN), lambda i: (i, 0)),
        ],
        out_shape=[
            jax.ShapeDtypeStruct((_NP, _D_IN), jnp.float32),
            jax.ShapeDtypeStruct((_NP, _D_IN), jnp.float32),
        ],
    )(x, d, d)


def _tc_b(p, xs, dis, W1, b1):
    def body(p0_ref, p1_ref, xs_ref, dis_ref, w_ref, b_ref, h0_ref, h1_ref):
        agg = dis_ref[...] * (p0_ref[...] + p1_ref[...] - xs_ref[...])
        h = jnp.dot(agg, w_ref[...], preferred_element_type=jnp.float32,
                    precision="highest") + b_ref[...]
        h = jnp.where(h >= 0.0, h, 0.01 * h)
        h0_ref[...] = h[:, :128] * dis_ref[...]
        h1_ref[...] = h[:, 128:] * dis_ref[...]

    nb = _NP // _BN
    return pl.pallas_call(
        body,
        grid=(nb,),
        in_specs=[
            pl.BlockSpec((_BN, 128), lambda i: (i, 0)),
            pl.BlockSpec((_BN, 128), lambda i: (i + nb, 0)),
            pl.BlockSpec((_BN, 128), lambda i: (i, 0)),
            pl.BlockSpec((_BN, 128), lambda i: (i, 0)),
            pl.BlockSpec((_D_IN, _D_HID), lambda i: (0, 0)),
            pl.BlockSpec((1, _D_HID), lambda i: (0, 0)),
        ],
        out_specs=[
            pl.BlockSpec((_BN, 128), lambda i: (i, 0)),
            pl.BlockSpec((_BN, 128), lambda i: (i, 0)),
        ],
        out_shape=[
            jax.ShapeDtypeStruct((_NP, 128), jnp.float32),
            jax.ShapeDtypeStruct((_NP, 128), jnp.float32),
        ],
    )(p, p, xs, dis, W1, b1)


def _tc_c(q, dis, Wmu, bmu, Wls, bls):
    def body(q0_ref, q1_ref, dis_ref, wmu_ref, bmu_ref, wls_ref, bls_ref,
             mu_ref, ls_ref):
        a0 = dis_ref[...] * q0_ref[...]
        a1 = dis_ref[...] * q1_ref[...]
        mu_ref[...] = (
            jnp.dot(a0, wmu_ref[:128, :], preferred_element_type=jnp.float32,
                    precision="highest")
            + jnp.dot(a1, wmu_ref[128:, :], preferred_element_type=jnp.float32,
                      precision="highest")
            + bmu_ref[...])
        ls_ref[...] = (
            jnp.dot(a0, wls_ref[:128, :], preferred_element_type=jnp.float32,
                    precision="highest")
            + jnp.dot(a1, wls_ref[128:, :], preferred_element_type=jnp.float32,
                      precision="highest")
            + bls_ref[...])

    nb = _NP // _BN
    return pl.pallas_call(
        body,
        grid=(nb,),
        in_specs=[
            pl.BlockSpec((_BN, 128), lambda i: (i, 0)),
            pl.BlockSpec((_BN, 128), lambda i: (i + nb, 0)),
            pl.BlockSpec((_BN, 128), lambda i: (i, 0)),
            pl.BlockSpec((_D_HID, _D_OUT), lambda i: (0, 0)),
            pl.BlockSpec((1, _D_OUT), lambda i: (0, 0)),
            pl.BlockSpec((_D_HID, _D_OUT), lambda i: (0, 0)),
            pl.BlockSpec((1, _D_OUT), lambda i: (0, 0)),
        ],
        out_specs=[
            pl.BlockSpec((_BN, _D_OUT), lambda i: (i, 0)),
            pl.BlockSpec((_BN, _D_OUT), lambda i: (i, 0)),
        ],
        out_shape=[
            jax.ShapeDtypeStruct((_N, _D_OUT), jnp.float32),
            jax.ShapeDtypeStruct((_N, _D_OUT), jnp.float32),
        ],
    )(q, q, dis, Wmu, bmu, Wls, bls)


def kernel(x, edge_index, W1, b1, Wmu, bmu, Wls, bls):
    src = edge_index[0].reshape(_R, _C)
    dst = edge_index[1].reshape(_R, _C)
    ones = jnp.ones((_NP, 16), jnp.float32)

    d = _sc_degree(dst, ones)
    xs, dis = _tc_a(x, d)
    p = _sc_pass1(xs, src, dst)
    h0, h1 = _tc_b(p, xs, dis, W1, b1.reshape(1, _D_HID))
    q = _sc_pass2(h0, h1, src, dst)
    mu, ls = _tc_c(q, dis, Wmu, bmu.reshape(1, _D_OUT),
                   Wls, bls.reshape(1, _D_OUT))
    return (mu, ls)


# speedup vs baseline: 25.3739x; 1.0188x over previous
import functools

import jax
import jax.numpy as jnp
from jax import lax
from jax.experimental import pallas as pl
from jax.experimental.pallas import tpu as pltpu
from jax.experimental.pallas import tpu_sc as plsc

_N = 10000
_E = 320000
_D_IN = 128
_D_OUT = 128
_D_HID = 256

_NC = 2
_NS = 16
_NP = 10240
_RPT = _NP // _NS

_C = 125
_R = _E // _C
_ESC = _E // _NC
_CH1 = _ESC // _NS // _C
_CH2 = _E // _NS // _C

_mesh = plsc.VectorSubcoreMesh(core_axis_name="c", subcore_axis_name="s")


@functools.partial(
    pl.kernel,
    out_type=jax.ShapeDtypeStruct((_NC * _NP, 16), jnp.float32),
    mesh=_mesh,
    scratch_types=[
        pltpu.VMEM((_CH1, _C), jnp.int32),
        pltpu.VMEM((128, 16), jnp.float32),
        pltpu.SemaphoreType.DMA,
        pltpu.VMEM_SHARED((_NP, 16), jnp.float32),
    ],
)
def _sc_degree(dst_hbm, ones_hbm, out_hbm, dst_v, ones_v, sem, acc):
    c = lax.axis_index("c")
    s = lax.axis_index("s")
    row0 = c * (_ESC // _C) + s * _CH1
    base = s * _RPT
    pltpu.sync_copy(dst_hbm.at[pl.ds(row0, _CH1)], dst_v)
    pltpu.sync_copy(ones_hbm.at[pl.ds(0, 128)], ones_v)
    pltpu.sync_copy(ones_hbm.at[pl.ds(base, _RPT)], acc.at[pl.ds(base, _RPT)])
    plsc.subcore_barrier()

    def issue(k):
        pltpu.async_copy(ones_v.at[pl.ds(0, _C)], acc.at[dst_v.at[k]], sem,
                         add=True)

    def wait_one():
        pltpu.make_async_copy(ones_v.at[pl.ds(0, _C)], acc.at[dst_v.at[0]],
                              sem).wait()

    _OUT = 4
    for k in range(_OUT):
        issue(k)

    def body(k, carry):
        wait_one()
        issue(k + _OUT)
        return carry

    lax.fori_loop(0, _CH1 - _OUT, body, 0)
    for _ in range(_OUT):
        wait_one()
    plsc.subcore_barrier()
    pltpu.sync_copy(acc.at[pl.ds(base, _RPT)],
                    out_hbm.at[pl.ds(c * _NP + base, _RPT)])


_CR = 40
_PAIRS = _CR // 2


def _run_rounds(tables, c, src_hbm, dst_hbm, src_v, dst_v, rowsA, rowsB,
                sga, sgb, ssa, ssb, acc, row0, nrounds):
    def issue_gather(k, buf, sem):
        if len(tables) == 1:
            pltpu.async_copy(tables[0].at[src_v.at[k]], buf, sem)
        else:
            @pl.when(c == 0)
            def _():
                pltpu.async_copy(tables[0].at[src_v.at[k]], buf, sem)

            @pl.when(c == 1)
            def _():
                pltpu.async_copy(tables[1].at[src_v.at[k]], buf, sem)

    def wait_gather(buf, sem):
        pltpu.make_async_copy(tables[0].at[src_v.at[0]], buf, sem).wait()

    def wait_scatter(buf, sem):
        pltpu.make_async_copy(buf, acc.at[dst_v.at[0]], sem).wait()

    for r in range(nrounds):
        base_r = row0 + r * _CR
        pltpu.sync_copy(src_hbm.at[pl.ds(base_r, _CR)], src_v)
        pltpu.sync_copy(dst_hbm.at[pl.ds(base_r, _CR)], dst_v)
        issue_gather(0, rowsA, sga)
        issue_gather(1, rowsB, sgb)

        def pair(j, carry):
            k0 = 2 * j
            wait_gather(rowsA, sga)
            pltpu.async_copy(rowsA, acc.at[dst_v.at[k0]], ssa, add=True)
            wait_gather(rowsB, sgb)
            pltpu.async_copy(rowsB, acc.at[dst_v.at[k0 + 1]], ssb, add=True)

            @pl.when(j < _PAIRS - 1)
            def _():
                wait_scatter(rowsA, ssa)
                issue_gather(k0 + 2, rowsA, sga)
                wait_scatter(rowsB, ssb)
                issue_gather(k0 + 3, rowsB, sgb)
            return carry

        lax.fori_loop(0, _PAIRS, pair, 0)
        wait_scatter(rowsA, ssa)
        wait_scatter(rowsB, ssb)


_PASS_SCRATCH = [
    pltpu.VMEM((_CR, _C), jnp.int32),
    pltpu.VMEM((_CR, _C), jnp.int32),
    pltpu.VMEM((_C, 128), jnp.float32),
    pltpu.VMEM((_C, 128), jnp.float32),
    pltpu.SemaphoreType.DMA,
    pltpu.SemaphoreType.DMA,
    pltpu.SemaphoreType.DMA,
    pltpu.SemaphoreType.DMA,
    pltpu.VMEM_SHARED((_NP, 128), jnp.float32),
]


@functools.partial(
    pl.kernel,
    out_type=jax.ShapeDtypeStruct((_NC * _NP, 128), jnp.float32),
    mesh=_mesh,
    scratch_types=_PASS_SCRATCH,
)
def _sc_pass1(xs_hbm, src_hbm, dst_hbm, out_hbm, src_v, dst_v, rowsA, rowsB,
              sga, sgb, ssa, ssb, acc):
    c = lax.axis_index("c")
    s = lax.axis_index("s")
    row0 = c * (_ESC // _C) + s * _CH1
    base = s * _RPT
    pltpu.sync_copy(xs_hbm.at[pl.ds(base, _RPT)], acc.at[pl.ds(base, _RPT)])
    plsc.subcore_barrier()
    _run_rounds((xs_hbm,), c, src_hbm, dst_hbm, src_v, dst_v, rowsA, rowsB,
                sga, sgb, ssa, ssb, acc, row0, _CH1 // _CR)
    plsc.subcore_barrier()
    pltpu.sync_copy(acc.at[pl.ds(base, _RPT)],
                    out_hbm.at[pl.ds(c * _NP + base, _RPT)])


@functools.partial(
    pl.kernel,
    out_type=jax.ShapeDtypeStruct((_NC * _NP, 128), jnp.float32),
    mesh=_mesh,
    scratch_types=_PASS_SCRATCH,
)
def _sc_pass2(h0_hbm, h1_hbm, src_hbm, dst_hbm, out_hbm, src_v, dst_v,
              rowsA, rowsB, sga, sgb, ssa, ssb, acc):
    c = lax.axis_index("c")
    s = lax.axis_index("s")
    row0 = s * _CH2
    base = s * _RPT

    @pl.when(c == 0)
    def _():
        pltpu.sync_copy(h0_hbm.at[pl.ds(base, _RPT)], acc.at[pl.ds(base, _RPT)])

    @pl.when(c == 1)
    def _():
        pltpu.sync_copy(h1_hbm.at[pl.ds(base, _RPT)], acc.at[pl.ds(base, _RPT)])

    plsc.subcore_barrier()
    _run_rounds((h0_hbm, h1_hbm), c, src_hbm, dst_hbm, src_v, dst_v,
                rowsA, rowsB, sga, sgb, ssa, ssb, acc, row0, _CH2 // _CR)
    plsc.subcore_barrier()
    pltpu.sync_copy(acc.at[pl.ds(base, _RPT)],
                    out_hbm.at[pl.ds(c * _NP + base, _RPT)])


_BN = 1024


def _tc_a(x, d):
    def body(x_ref, d0_ref, d1_ref, xs_ref, dis_ref):
        deg = d0_ref[:, 0:1] + d1_ref[:, 0:1] - 1.0
        dis = 1.0 / jnp.sqrt(deg)
        xs_ref[...] = x_ref[...] * dis
        dis_ref[...] = jnp.broadcast_to(dis, dis_ref.shape)

    nb = _NP // _BN
    return pl.pallas_call(
        body,
        grid=(nb,),
        in_specs=[
            pl.BlockSpec((_BN, _D_IN), lambda i: (i, 0)),
            pl.BlockSpec((_BN, 16), lambda i: (i, 0)),
            pl.BlockSpec((_BN, 16), lambda i: (i + nb, 0)),
        ],
        out_specs=[
            pl.BlockSpec((_BN, _D_IN), lambda i: (i, 0)),
            pl.BlockSpec((_BN, _D_IN), lambda i: (i, 0)),
        ],
        out_shape=[
            jax.ShapeDtypeStruct((_NP, _D_IN), jnp.float32),
            jax.ShapeDtypeStruct((_NP, _D_IN), jnp.float32),
        ],
    )(x, d, d)


def _tc_b(p, xs, dis, W1, b1):
    def body(p0_ref, p1_ref, xs_ref, dis_ref, w_ref, b_ref, h0_ref, h1_ref):
        agg = dis_ref[...] * (p0_ref[...] + p1_ref[...] - xs_ref[...])
        h = jnp.dot(agg, w_ref[...], preferred_element_type=jnp.float32) + b_ref[...]
        h = jnp.where(h >= 0.0, h, 0.01 * h)
        h0_ref[...] = h[:, :128] * dis_ref[...]
        h1_ref[...] = h[:, 128:] * dis_ref[...]

    nb = _NP // _BN
    return pl.pallas_call(
        body,
        grid=(nb,),
        in_specs=[
            pl.BlockSpec((_BN, 128), lambda i: (i, 0)),
            pl.BlockSpec((_BN, 128), lambda i: (i + nb, 0)),
            pl.BlockSpec((_BN, 128), lambda i: (i, 0)),
            pl.BlockSpec((_BN, 128), lambda i: (i, 0)),
            pl.BlockSpec((_D_IN, _D_HID), lambda i: (0, 0)),
            pl.BlockSpec((1, _D_HID), lambda i: (0, 0)),
        ],
        out_specs=[
            pl.BlockSpec((_BN, 128), lambda i: (i, 0)),
            pl.BlockSpec((_BN, 128), lambda i: (i, 0)),
        ],
        out_shape=[
            jax.ShapeDtypeStruct((_NP, 128), jnp.float32),
            jax.ShapeDtypeStruct((_NP, 128), jnp.float32),
        ],
    )(p, p, xs, dis, W1, b1)


def _tc_c(q, dis, Wmu, bmu, Wls, bls):
    def body(q0_ref, q1_ref, dis_ref, wmu_ref, bmu_ref, wls_ref, bls_ref,
             mu_ref, ls_ref):
        a0 = dis_ref[...] * q0_ref[...]
        a1 = dis_ref[...] * q1_ref[...]
        mu_ref[...] = (
            jnp.dot(a0, wmu_ref[:128, :], preferred_element_type=jnp.float32)
            + jnp.dot(a1, wmu_ref[128:, :], preferred_element_type=jnp.float32)
            + bmu_ref[...])
        ls_ref[...] = (
            jnp.dot(a0, wls_ref[:128, :], preferred_element_type=jnp.float32)
            + jnp.dot(a1, wls_ref[128:, :], preferred_element_type=jnp.float32)
            + bls_ref[...])

    nb = _NP // _BN
    return pl.pallas_call(
        body,
        grid=(nb,),
        in_specs=[
            pl.BlockSpec((_BN, 128), lambda i: (i, 0)),
            pl.BlockSpec((_BN, 128), lambda i: (i + nb, 0)),
            pl.BlockSpec((_BN, 128), lambda i: (i, 0)),
            pl.BlockSpec((_D_HID, _D_OUT), lambda i: (0, 0)),
            pl.BlockSpec((1, _D_OUT), lambda i: (0, 0)),
            pl.BlockSpec((_D_HID, _D_OUT), lambda i: (0, 0)),
            pl.BlockSpec((1, _D_OUT), lambda i: (0, 0)),
        ],
        out_specs=[
            pl.BlockSpec((_BN, _D_OUT), lambda i: (i, 0)),
            pl.BlockSpec((_BN, _D_OUT), lambda i: (i, 0)),
        ],
        out_shape=[
            jax.ShapeDtypeStruct((_N, _D_OUT), jnp.float32),
            jax.ShapeDtypeStruct((_N, _D_OUT), jnp.float32),
        ],
    )(q, q, dis, Wmu, bmu, Wls, bls)


def kernel(x, edge_index, W1, b1, Wmu, bmu, Wls, bls):
    src = edge_index[0].reshape(_R, _C)
    dst = edge_index[1].reshape(_R, _C)
    ones = jnp.ones((_NP, 16), jnp.float32)

    d = _sc_degree(dst, ones)
    xs, dis = _tc_a(x, d)
    p = _sc_pass1(xs, src, dst)
    h0, h1 = _tc_b(p, xs, dis, W1, b1.reshape(1, _D_HID))
    q = _sc_pass2(h0, h1, src, dst)
    mu, ls = _tc_c(q, dis, Wmu, bmu.reshape(1, _D_OUT),
                   Wls, bls.reshape(1, _D_OUT))
    return (mu, ls)
